# Initial kernel scaffold; baseline (speedup 1.0000x reference)
import jax
import jax.numpy as jnp
from jax.experimental import pallas as pl


def kernel(S, Y, y_pred):
    raise NotImplementedError("write your pallas kernel here")


# speedup vs baseline: 282.0842x; 282.0842x over previous
import functools

import jax
import jax.numpy as jnp
from jax import lax
from jax.experimental import pallas as pl
from jax.experimental.pallas import tpu as pltpu
from jax.experimental.pallas import tpu_sc as plsc

_N = 1048576
_NC, _NS, _L = 2, 16, 16
_NW = _NC * _NS
_CHUNK = _N // _NW
_ALPHA = 0.1

_mesh = plsc.VectorSubcoreMesh(core_axis_name="c", subcore_axis_name="s")


@functools.partial(
    pl.kernel,
    mesh=_mesh,
    compiler_params=pltpu.CompilerParams(needs_layout_passes=False),
    out_type=jax.ShapeDtypeStruct((_NW, _L), jnp.float32),
    scratch_types=[
        pltpu.VMEM((_CHUNK,), jnp.float32),
        pltpu.VMEM((_CHUNK,), jnp.int32),
        pltpu.VMEM((128,), jnp.float32),
        pltpu.VMEM((_L,), jnp.float32),
        pltpu.SemaphoreType.DMA,
        pltpu.SemaphoreType.DMA,
        pltpu.SemaphoreType.DMA,
    ],
)
def _pinball_partials(s_hbm, y_hbm, t_hbm, out_hbm,
                      s_v, y_v, t_v, o_v, sem_s, sem_y, sem_t):
    wid = lax.axis_index("s") * _NC + lax.axis_index("c")
    base = wid * _CHUNK
    cp_t = pltpu.async_copy(t_hbm, t_v, sem_t)
    cp_s = pltpu.async_copy(s_hbm.at[pl.ds(base, _CHUNK)], s_v, sem_s)
    cp_y = pltpu.async_copy(y_hbm.at[pl.ds(base, _CHUNK)], y_v, sem_y)
    cp_t.wait()
    cp_s.wait()
    cp_y.wait()

    def step(i, acc):
        s = s_v[pl.ds(i * _L, _L)]
        idx = y_v[pl.ds(i * _L, _L)]
        q = plsc.load_gather(t_v, [idx])
        d = q - s
        return acc + jnp.maximum((1.0 - _ALPHA) * d, -_ALPHA * d)

    acc = lax.fori_loop(0, _CHUNK // _L, step, jnp.zeros((_L,), jnp.float32))
    o_v[...] = acc
    pltpu.sync_copy(o_v, out_hbm.at[wid])


def kernel(S, Y, y_pred):
    table = jnp.zeros((128,), jnp.float32).at[:100].set(y_pred)
    partials = _pinball_partials(S, Y.astype(jnp.int32), table)
    return jnp.sum(partials) / _N


# speedup vs baseline: 357.2195x; 1.2664x over previous
import functools

import jax
import jax.numpy as jnp
from jax import lax
from jax.experimental import pallas as pl
from jax.experimental.pallas import tpu as pltpu
from jax.experimental.pallas import tpu_sc as plsc

_N = 1048576
_NC, _NS, _L = 2, 16, 16
_NW = _NC * _NS
_CHUNK = _N // _NW
_ALPHA = 0.1

_mesh = plsc.VectorSubcoreMesh(core_axis_name="c", subcore_axis_name="s")


@functools.partial(
    pl.kernel,
    mesh=_mesh,
    compiler_params=pltpu.CompilerParams(needs_layout_passes=False),
    out_type=jax.ShapeDtypeStruct((_NW, _L), jnp.float32),
    scratch_types=[
        pltpu.VMEM((_CHUNK,), jnp.float32),
        pltpu.VMEM((_CHUNK,), jnp.int32),
        pltpu.VMEM((128,), jnp.float32),
        pltpu.VMEM((_L,), jnp.float32),
        pltpu.SemaphoreType.DMA,
        pltpu.SemaphoreType.DMA,
        pltpu.SemaphoreType.DMA,
    ],
)
def _pinball_partials(s_hbm, y_hbm, t_hbm, out_hbm,
                      s_v, y_v, t_v, o_v, sem_s, sem_y, sem_t):
    wid = lax.axis_index("s") * _NC + lax.axis_index("c")
    base = wid * _CHUNK
    cp_t = pltpu.async_copy(t_hbm, t_v, sem_t)
    cp_s = pltpu.async_copy(s_hbm.at[pl.ds(base, _CHUNK)], s_v, sem_s)
    cp_y = pltpu.async_copy(y_hbm.at[pl.ds(base, _CHUNK)], y_v, sem_y)
    cp_t.wait()
    cp_s.wait()
    cp_y.wait()

    def step(i, acc):
        s = s_v[pl.ds(i, _L)]
        idx = y_v[pl.ds(i, _L)]
        q = plsc.load_gather(t_v, [idx])
        d = q - s
        return acc + jnp.maximum((1.0 - _ALPHA) * d, -_ALPHA * d)

    acc = plsc.parallel_loop(
        0, _CHUNK, _L, unroll=8, carry=jnp.zeros((_L,), jnp.float32))(step)
    o_v[...] = acc
    pltpu.sync_copy(o_v, out_hbm.at[wid])


def kernel(S, Y, y_pred):
    table = jnp.zeros((128,), jnp.float32).at[:100].set(y_pred)
    partials = _pinball_partials(S, Y.astype(jnp.int32), table)
    return jnp.sum(partials) / _N


# speedup vs baseline: 357.9404x; 1.0020x over previous
import functools

import jax
import jax.numpy as jnp
from jax import lax
from jax.experimental import pallas as pl
from jax.experimental.pallas import tpu as pltpu
from jax.experimental.pallas import tpu_sc as plsc

_N = 1048576
_NC, _NS, _L = 2, 16, 16
_NW = _NC * _NS
_CHUNK = _N // _NW
_ALPHA = 0.1

_mesh = plsc.VectorSubcoreMesh(core_axis_name="c", subcore_axis_name="s")


@functools.partial(
    pl.kernel,
    mesh=_mesh,
    compiler_params=pltpu.CompilerParams(needs_layout_passes=False),
    out_type=jax.ShapeDtypeStruct((_NW, _L), jnp.float32),
    scratch_types=[
        pltpu.VMEM((_CHUNK,), jnp.float32),
        pltpu.VMEM((_CHUNK,), jnp.int32),
        pltpu.VMEM((100,), jnp.float32),
        pltpu.VMEM((_L,), jnp.float32),
        pltpu.SemaphoreType.DMA,
        pltpu.SemaphoreType.DMA,
        pltpu.SemaphoreType.DMA,
    ],
)
def _pinball_partials(s_hbm, y_hbm, t_hbm, out_hbm,
                      s_v, y_v, t_v, o_v, sem_s, sem_y, sem_t):
    wid = lax.axis_index("s") * _NC + lax.axis_index("c")
    base = wid * _CHUNK
    cp_t = pltpu.async_copy(t_hbm, t_v, sem_t)
    cp_s = pltpu.async_copy(s_hbm.at[pl.ds(base, _CHUNK)], s_v, sem_s)
    cp_y = pltpu.async_copy(y_hbm.at[pl.ds(base, _CHUNK)], y_v, sem_y)
    cp_t.wait()
    cp_s.wait()
    cp_y.wait()

    def step(i, accs):
        a0, a1 = accs
        s0 = s_v[pl.ds(i, _L)]
        idx0 = y_v[pl.ds(i, _L)]
        s1 = s_v[pl.ds(i + _L, _L)]
        idx1 = y_v[pl.ds(i + _L, _L)]
        d0 = plsc.load_gather(t_v, [idx0]) - s0
        d1 = plsc.load_gather(t_v, [idx1]) - s1
        a0 = a0 + jnp.maximum((1.0 - _ALPHA) * d0, -_ALPHA * d0)
        a1 = a1 + jnp.maximum((1.0 - _ALPHA) * d1, -_ALPHA * d1)
        return a0, a1

    z = jnp.zeros((_L,), jnp.float32)
    a0, a1 = plsc.parallel_loop(
        0, _CHUNK, 2 * _L, unroll=8, carry=(z, z))(step)
    o_v[...] = a0 + a1
    pltpu.sync_copy(o_v, out_hbm.at[wid])


def kernel(S, Y, y_pred):
    partials = _pinball_partials(S, Y.astype(jnp.int32), y_pred)
    return jnp.sum(partials) / _N
